# Initial kernel scaffold; baseline (speedup 1.0000x reference)
import jax
import jax.numpy as jnp
from jax.experimental import pallas as pl


def kernel(x, edge_src0, edge_dst0, edge_src1, edge_dst1, num_dst0, num_dst1, W_self1, W_neigh1, b1, W_self2, W_neigh2, b2):
    raise NotImplementedError("write your pallas kernel here")


# speedup vs baseline: 4.1341x; 4.1341x over previous
import functools

import jax
import jax.numpy as jnp
from jax import lax
from jax.experimental import pallas as pl
from jax.experimental.pallas import tpu as pltpu
from jax.experimental.pallas import tpu_sc as plsc

N0, N1, N2 = 10000, 2048, 512
E0, E1 = 32768, 8192
D = 512

_NC, _NS = 2, 16
_CHUNK = 128
_SW = 128
_STRIPS = D // _SW
_SPC = _STRIPS // _NC
_ZR = 128


def _make_agg(n_dst, n_edges):
    e_per_w = n_edges // _NS
    n_chunks = e_per_w // _CHUNK
    vpc = _CHUNK // 16
    rdpt = n_dst // _NS
    frpt = n_dst * _SPC // _NS
    assert n_chunks * _CHUNK * _NS == n_edges
    assert rdpt * _NS == n_dst

    mesh = plsc.VectorSubcoreMesh(core_axis_name="c", subcore_axis_name="s",
                                  num_cores=_NC, num_subcores=_NS)

    @functools.partial(
        pl.kernel,
        out_type=(
            jax.ShapeDtypeStruct((_NC, n_dst * _SPC, _SW), jnp.float32),
            jax.ShapeDtypeStruct((_NC * n_dst, 128), jnp.float32),
        ),
        mesh=mesh,
        scratch_types=[
            pltpu.VMEM((n_chunks, _CHUNK), jnp.int32),
            pltpu.VMEM((n_chunks, _CHUNK), jnp.int32),
            pltpu.VMEM((_SPC, _CHUNK), jnp.int32),
            pltpu.VMEM((_CHUNK,), jnp.int32),
            pltpu.VMEM((_SPC, _CHUNK, _SW), jnp.float32),
            pltpu.VMEM((_CHUNK, _SW), jnp.float32),
            pltpu.VMEM((_ZR, _SW), jnp.float32),
            pltpu.VMEM_SHARED((n_dst * _SPC, _SW), jnp.float32),
            pltpu.VMEM_SHARED((n_dst, _SW), jnp.float32),
            pltpu.SemaphoreType.DMA,
        ],
    )
    def agg_kernel(src_hbm, dst_hbm, x_hbm, agg_out, deg_out,
                   src_v, dst_v, gidx_v, sidx_v, rows_v, ones_v, zbuf_v,
                   acc_sh, deg_sh, sem):
        c = lax.axis_index("c")
        s = lax.axis_index("s")

        zero = jnp.zeros((16,), jnp.float32)
        one = jnp.ones((16,), jnp.float32)
        vpr = _SW // 16

        def fill_z(i, _):
            zbuf_v[i // vpr, pl.ds((i % vpr) * 16, 16)] = zero
            return 0
        lax.fori_loop(0, _ZR * vpr, fill_z, 0)

        pltpu.sync_copy(src_hbm.at[pl.ds(s * n_chunks, n_chunks)], src_v)
        pltpu.sync_copy(dst_hbm.at[pl.ds(s * n_chunks, n_chunks)], dst_v)

        for t in range(0, frpt, _ZR):
            w = min(_ZR, frpt - t)
            pltpu.sync_copy(zbuf_v.at[pl.ds(0, w)],
                            acc_sh.at[pl.ds(s * frpt + t, w)])
        for t in range(0, rdpt, _ZR):
            w = min(_ZR, rdpt - t)
            pltpu.sync_copy(zbuf_v.at[pl.ds(0, w)],
                            deg_sh.at[pl.ds(s * rdpt + t, w)])

        def fill_o(i, _):
            ones_v[i // vpr, pl.ds((i % vpr) * 16, 16)] = one
            return 0
        lax.fori_loop(0, _CHUNK * vpr, fill_o, 0)

        plsc.subcore_barrier()

        strip0 = (c * _SPC).astype(jnp.int32)

        def chunk_body(j, _):
            copies = []
            for b in range(_SPC):
                for v in range(vpc):
                    gidx_v[b, pl.ds(v * 16, 16)] = (
                        src_v[j, pl.ds(v * 16, 16)] * _STRIPS + strip0 + b)
                copies.append(
                    pltpu.async_copy(x_hbm.at[gidx_v.at[b]], rows_v.at[b], sem))
            for b in range(_SPC):
                copies[b].wait()
                for v in range(vpc):
                    sidx_v[pl.ds(v * 16, 16)] = (
                        dst_v[j, pl.ds(v * 16, 16)] * _SPC + b)
                pltpu.sync_copy(rows_v.at[b], acc_sh.at[sidx_v], add=True)
            for v in range(vpc):
                sidx_v[pl.ds(v * 16, 16)] = dst_v[j, pl.ds(v * 16, 16)]
            pltpu.sync_copy(ones_v, deg_sh.at[sidx_v], add=True)
            return 0
        lax.fori_loop(0, n_chunks, chunk_body, 0)

        plsc.subcore_barrier()

        pltpu.sync_copy(acc_sh.at[pl.ds(s * frpt, frpt)],
                        agg_out.at[c].at[pl.ds(s * frpt, frpt)])
        pltpu.sync_copy(deg_sh.at[pl.ds(s * rdpt, rdpt)],
                        deg_out.at[pl.ds(c * n_dst + s * rdpt, rdpt)])

    return agg_kernel


def _sage_mm(m, relu):
    def body(dst_ref, aggl_ref, aggr_ref, deg_ref, ws_ref, wnt_ref, wnb_ref,
             b_ref, out_ref):
        deg = jnp.maximum(deg_ref[:, 0:1], 1.0)
        acc = jnp.dot(dst_ref[...], ws_ref[...],
                      preferred_element_type=jnp.float32)
        acc = acc + jnp.dot(aggl_ref[...] / deg, wnt_ref[...],
                            preferred_element_type=jnp.float32)
        acc = acc + jnp.dot(aggr_ref[...] / deg, wnb_ref[...],
                            preferred_element_type=jnp.float32)
        acc = acc + b_ref[...]
        if relu:
            acc = jnp.maximum(acc, 0.0)
        out_ref[...] = acc

    half = D // _NC
    return pl.pallas_call(
        body,
        out_shape=jax.ShapeDtypeStruct((m, D), jnp.float32),
        grid=(1,),
        in_specs=[
            pl.BlockSpec((m, D), lambda i: (0, 0)),
            pl.BlockSpec((m, half), lambda i: (0, 0)),
            pl.BlockSpec((m, half), lambda i: (0, 0)),
            pl.BlockSpec((m, 128), lambda i: (0, 0)),
            pl.BlockSpec((D, D), lambda i: (0, 0)),
            pl.BlockSpec((half, D), lambda i: (0, 0)),
            pl.BlockSpec((half, D), lambda i: (0, 0)),
            pl.BlockSpec((1, D), lambda i: (0, 0)),
        ],
        out_specs=pl.BlockSpec((m, D), lambda i: (0, 0)),
    )


_agg0 = _make_agg(N1, E0)
_agg1 = _make_agg(N2, E1)
_mm1 = _sage_mm(N1, relu=True)
_mm2 = _sage_mm(N2, relu=False)


def kernel(x, edge_src0, edge_dst0, edge_src1, edge_dst1, num_dst0, num_dst1,
           W_self1, W_neigh1, b1, W_self2, W_neigh2, b2):
    src0 = edge_src0.astype(jnp.int32).reshape(E0 // _CHUNK, _CHUNK)
    dst0 = edge_dst0.astype(jnp.int32).reshape(E0 // _CHUNK, _CHUNK)
    src1 = edge_src1.astype(jnp.int32).reshape(E1 // _CHUNK, _CHUNK)
    dst1 = edge_dst1.astype(jnp.int32).reshape(E1 // _CHUNK, _CHUNK)
    half = D // _NC

    agg1, deg1 = _agg0(src0, dst0, x.reshape(N0 * _STRIPS, _SW))
    h = _mm1(x, agg1[0].reshape(N1, half), agg1[1].reshape(N1, half),
             deg1[:N1], W_self1, W_neigh1[:half], W_neigh1[half:],
             b1.reshape(1, D))
    agg2, deg2 = _agg1(src1, dst1, h.reshape(N1 * _STRIPS, _SW))
    return _mm2(h, agg2[0].reshape(N2, half), agg2[1].reshape(N2, half),
                deg2[:N2], W_self2, W_neigh2[:half], W_neigh2[half:],
                b2.reshape(1, D))


# speedup vs baseline: 4.7625x; 1.1520x over previous
import functools

import jax
import jax.numpy as jnp
from jax import lax
from jax.experimental import pallas as pl
from jax.experimental.pallas import tpu as pltpu
from jax.experimental.pallas import tpu_sc as plsc

N0, N1, N2 = 10000, 2048, 512
E0, E1 = 32768, 8192
D = 512

_NC, _NS = 2, 16
_CHUNK = 128
_SW = 128
_STRIPS = D // _SW
_SPC = _STRIPS // _NC
_ZR = 64
_DW = 128


def _make_agg(n_dst, n_edges):
    e_per_w = n_edges // _NS
    n_chunks = e_per_w // _CHUNK
    vpc = _CHUNK // 16
    rdpt = n_dst // _NS
    frpt = n_dst * _SPC // _NS
    assert n_chunks * _CHUNK * _NS == n_edges
    assert rdpt * _NS == n_dst

    mesh = plsc.VectorSubcoreMesh(core_axis_name="c", subcore_axis_name="s",
                                  num_cores=_NC, num_subcores=_NS)

    @functools.partial(
        pl.kernel,
        out_type=(
            jax.ShapeDtypeStruct((_NC, n_dst * _SPC, _SW), jnp.float32),
            jax.ShapeDtypeStruct((_NC * n_dst, _DW), jnp.float32),
        ),
        mesh=mesh,
        scratch_types=[
            pltpu.VMEM((n_chunks, _CHUNK), jnp.int32),
            pltpu.VMEM((n_chunks, _CHUNK), jnp.int32),
            pltpu.VMEM((_SPC, _CHUNK), jnp.int32),
            pltpu.VMEM((_CHUNK,), jnp.int32),
            pltpu.VMEM((_SPC, _CHUNK, _SW), jnp.float32),
            pltpu.VMEM((_CHUNK, _DW), jnp.float32),
            pltpu.VMEM((_ZR, _SW), jnp.float32),
            pltpu.VMEM((_ZR, _DW), jnp.float32),
            pltpu.VMEM_SHARED((n_dst * _SPC, _SW), jnp.float32),
            pltpu.VMEM_SHARED((n_dst, _DW), jnp.float32),
            pltpu.SemaphoreType.DMA,
            pltpu.SemaphoreType.DMA,
        ],
    )
    def agg_kernel(src_hbm, dst_hbm, x_hbm, agg_out, deg_out,
                   src_v, dst_v, gidx_v, sidx_v, rows_v, ones_v, zbuf_v,
                   zdeg_v, acc_sh, deg_sh, sem0, sem1):
        c = lax.axis_index("c")
        s = lax.axis_index("s")

        zero = jnp.zeros((16,), jnp.float32)
        one = jnp.ones((16,), jnp.float32)
        vpr = _SW // 16

        def fill_z(i, _):
            zbuf_v[i // vpr, pl.ds((i % vpr) * 16, 16)] = zero
            return 0
        lax.fori_loop(0, _ZR * vpr, fill_z, 0)

        def fill_zd(i, _):
            zdeg_v[i, pl.ds(0, 16)] = zero
            return 0
        lax.fori_loop(0, _ZR, fill_zd, 0)

        pltpu.sync_copy(src_hbm.at[pl.ds(s * n_chunks, n_chunks)], src_v)
        pltpu.sync_copy(dst_hbm.at[pl.ds(s * n_chunks, n_chunks)], dst_v)

        for t in range(0, frpt, _ZR):
            w = min(_ZR, frpt - t)
            pltpu.sync_copy(zbuf_v.at[pl.ds(0, w)],
                            acc_sh.at[pl.ds(s * frpt + t, w)])
        for t in range(0, rdpt, _ZR):
            w = min(_ZR, rdpt - t)
            pltpu.sync_copy(zdeg_v.at[pl.ds(0, w)],
                            deg_sh.at[pl.ds(s * rdpt + t, w)])

        def fill_o(i, _):
            ones_v[i, pl.ds(0, 16)] = one
            return 0
        lax.fori_loop(0, _CHUNK, fill_o, 0)

        plsc.subcore_barrier()

        strip0 = (c * _SPC).astype(jnp.int32)
        sems = (sem0, sem1)

        def stage_strip(b, j):
            for v in range(vpc):
                gidx_v[b, pl.ds(v * 16, 16)] = (
                    src_v[j, pl.ds(v * 16, 16)] * _STRIPS + strip0 + b)
            pltpu.async_copy(x_hbm.at[gidx_v.at[b]], rows_v.at[b], sems[b])

        def drain_strip(b, j):
            pltpu.make_async_copy(x_hbm.at[pl.ds(0, _CHUNK)],
                                  rows_v.at[b], sems[b]).wait()
            for v in range(vpc):
                sidx_v[pl.ds(v * 16, 16)] = (
                    dst_v[j, pl.ds(v * 16, 16)] * _SPC + b)
            pltpu.sync_copy(rows_v.at[b], acc_sh.at[sidx_v], add=True)

        def deg_scatter(j):
            for v in range(vpc):
                sidx_v[pl.ds(v * 16, 16)] = dst_v[j, pl.ds(v * 16, 16)]
            pltpu.sync_copy(ones_v, deg_sh.at[sidx_v], add=True)

        for b in range(_SPC):
            stage_strip(b, 0)

        def chunk_body(j, _):
            drain_strip(0, j)
            stage_strip(0, j + 1)
            drain_strip(1, j)
            stage_strip(1, j + 1)
            deg_scatter(j)
            return 0
        lax.fori_loop(0, n_chunks - 1, chunk_body, 0)
        for b in range(_SPC):
            drain_strip(b, n_chunks - 1)
        deg_scatter(n_chunks - 1)

        plsc.subcore_barrier()

        pltpu.sync_copy(acc_sh.at[pl.ds(s * frpt, frpt)],
                        agg_out.at[c].at[pl.ds(s * frpt, frpt)])
        pltpu.sync_copy(deg_sh.at[pl.ds(s * rdpt, rdpt)],
                        deg_out.at[pl.ds(c * n_dst + s * rdpt, rdpt)])

    return agg_kernel


def _sage_mm(m, relu):
    def body(dst_ref, aggl_ref, aggr_ref, deg_ref, ws_ref, wnt_ref, wnb_ref,
             b_ref, out_ref):
        deg = jnp.maximum(deg_ref[:, 0:1], 1.0)
        acc = jnp.dot(dst_ref[...], ws_ref[...],
                      preferred_element_type=jnp.float32)
        acc = acc + jnp.dot(aggl_ref[...] / deg, wnt_ref[...],
                            preferred_element_type=jnp.float32)
        acc = acc + jnp.dot(aggr_ref[...] / deg, wnb_ref[...],
                            preferred_element_type=jnp.float32)
        acc = acc + b_ref[...]
        if relu:
            acc = jnp.maximum(acc, 0.0)
        out_ref[...] = acc

    half = D // _NC
    return pl.pallas_call(
        body,
        out_shape=jax.ShapeDtypeStruct((m, D), jnp.float32),
        grid=(1,),
        in_specs=[
            pl.BlockSpec((m, D), lambda i: (0, 0)),
            pl.BlockSpec((m, half), lambda i: (0, 0)),
            pl.BlockSpec((m, half), lambda i: (0, 0)),
            pl.BlockSpec((m, _DW), lambda i: (0, 0)),
            pl.BlockSpec((D, D), lambda i: (0, 0)),
            pl.BlockSpec((half, D), lambda i: (0, 0)),
            pl.BlockSpec((half, D), lambda i: (0, 0)),
            pl.BlockSpec((1, D), lambda i: (0, 0)),
        ],
        out_specs=pl.BlockSpec((m, D), lambda i: (0, 0)),
    )


_agg0 = _make_agg(N1, E0)
_agg1 = _make_agg(N2, E1)
_mm1 = _sage_mm(N1, relu=True)
_mm2 = _sage_mm(N2, relu=False)


def kernel(x, edge_src0, edge_dst0, edge_src1, edge_dst1, num_dst0, num_dst1,
           W_self1, W_neigh1, b1, W_self2, W_neigh2, b2):
    src0 = edge_src0.astype(jnp.int32).reshape(E0 // _CHUNK, _CHUNK)
    dst0 = edge_dst0.astype(jnp.int32).reshape(E0 // _CHUNK, _CHUNK)
    src1 = edge_src1.astype(jnp.int32).reshape(E1 // _CHUNK, _CHUNK)
    dst1 = edge_dst1.astype(jnp.int32).reshape(E1 // _CHUNK, _CHUNK)
    half = D // _NC

    agg1, deg1 = _agg0(src0, dst0, x.reshape(N0 * _STRIPS, _SW))
    h = _mm1(x, agg1[0].reshape(N1, half), agg1[1].reshape(N1, half),
             deg1[:N1], W_self1, W_neigh1[:half], W_neigh1[half:],
             b1.reshape(1, D))
    agg2, deg2 = _agg1(src1, dst1, h.reshape(N1 * _STRIPS, _SW))
    return _mm2(h, agg2[0].reshape(N2, half), agg2[1].reshape(N2, half),
                deg2[:N2], W_self2, W_neigh2[:half], W_neigh2[half:],
                b2.reshape(1, D))


# speedup vs baseline: 4.7637x; 1.0003x over previous
import functools

import jax
import jax.numpy as jnp
from jax import lax
from jax.experimental import pallas as pl
from jax.experimental.pallas import tpu as pltpu
from jax.experimental.pallas import tpu_sc as plsc

N0, N1, N2 = 10000, 2048, 512
E0, E1 = 32768, 8192
D = 512

_NC, _NS = 2, 16
_CHUNK = 128
_SW = 128
_STRIPS = D // _SW
_SPC = _STRIPS // _NC
_ZR = 64
_DW = 128


def _make_agg(n_dst, n_edges):
    e_per_w = n_edges // _NS
    n_chunks = e_per_w // _CHUNK
    vpc = _CHUNK // 16
    rdpt = n_dst // _NS
    frpt = n_dst * _SPC // _NS
    assert n_chunks * _CHUNK * _NS == n_edges
    assert rdpt * _NS == n_dst

    mesh = plsc.VectorSubcoreMesh(core_axis_name="c", subcore_axis_name="s",
                                  num_cores=_NC, num_subcores=_NS)

    @functools.partial(
        pl.kernel,
        out_type=(
            jax.ShapeDtypeStruct((_NC, n_dst * _SPC, _SW), jnp.float32),
            jax.ShapeDtypeStruct((_NC * n_dst, _DW), jnp.float32),
        ),
        mesh=mesh,
        scratch_types=[
            pltpu.VMEM((n_chunks, _CHUNK), jnp.int32),
            pltpu.VMEM((n_chunks, _CHUNK), jnp.int32),
            pltpu.VMEM((_SPC, _CHUNK), jnp.int32),
            pltpu.VMEM((_CHUNK,), jnp.int32),
            pltpu.VMEM((_SPC, _CHUNK, _SW), jnp.float32),
            pltpu.VMEM((_CHUNK, _DW), jnp.float32),
            pltpu.VMEM((_ZR, _SW), jnp.float32),
            pltpu.VMEM((_ZR, _DW), jnp.float32),
            pltpu.VMEM_SHARED((n_dst * _SPC, _SW), jnp.float32),
            pltpu.VMEM_SHARED((n_dst, _DW), jnp.float32),
            pltpu.SemaphoreType.DMA,
            pltpu.SemaphoreType.DMA,
        ],
    )
    def agg_kernel(src_hbm, dst_hbm, x_hbm, agg_out, deg_out,
                   src_v, dst_v, gidx_v, sidx_v, rows_v, ones_v, zbuf_v,
                   zdeg_v, acc_sh, deg_sh, sem0, sem1):
        c = lax.axis_index("c")
        s = lax.axis_index("s")

        zero = jnp.zeros((16,), jnp.float32)
        one = jnp.ones((16,), jnp.float32)
        vpr = _SW // 16

        def fill_z(i, _):
            zbuf_v[i // vpr, pl.ds((i % vpr) * 16, 16)] = zero
            return 0
        lax.fori_loop(0, _ZR * vpr, fill_z, 0)

        def fill_zd(i, _):
            zdeg_v[i, pl.ds(0, 16)] = zero
            return 0
        lax.fori_loop(0, _ZR, fill_zd, 0)

        pltpu.sync_copy(src_hbm.at[pl.ds(s * n_chunks, n_chunks)], src_v)
        pltpu.sync_copy(dst_hbm.at[pl.ds(s * n_chunks, n_chunks)], dst_v)

        for t in range(0, frpt, _ZR):
            w = min(_ZR, frpt - t)
            pltpu.sync_copy(zbuf_v.at[pl.ds(0, w)],
                            acc_sh.at[pl.ds(s * frpt + t, w)])
        for t in range(0, rdpt, _ZR):
            w = min(_ZR, rdpt - t)
            pltpu.sync_copy(zdeg_v.at[pl.ds(0, w)],
                            deg_sh.at[pl.ds(s * rdpt + t, w)])

        def fill_o(i, _):
            ones_v[i, pl.ds(0, 16)] = one
            return 0
        lax.fori_loop(0, _CHUNK, fill_o, 0)

        plsc.subcore_barrier()

        strip0 = (c * _SPC).astype(jnp.int32)
        sems = (sem0, sem1)

        def stage_strip(b, j):
            for v in range(vpc):
                gidx_v[b, pl.ds(v * 16, 16)] = (
                    src_v[j, pl.ds(v * 16, 16)] * _STRIPS + strip0 + b)
            pltpu.async_copy(x_hbm.at[gidx_v.at[b]], rows_v.at[b], sems[b])

        def drain_strip(b, j):
            pltpu.make_async_copy(x_hbm.at[pl.ds(0, _CHUNK)],
                                  rows_v.at[b], sems[b]).wait()
            for v in range(vpc):
                sidx_v[pl.ds(v * 16, 16)] = (
                    dst_v[j, pl.ds(v * 16, 16)] * _SPC + b)
            pltpu.sync_copy(rows_v.at[b], acc_sh.at[sidx_v], add=True)

        def deg_scatter(j):
            for v in range(vpc):
                sidx_v[pl.ds(v * 16, 16)] = dst_v[j, pl.ds(v * 16, 16)]
            pltpu.sync_copy(ones_v, deg_sh.at[sidx_v], add=True)

        for b in range(_SPC):
            stage_strip(b, 0)

        def chunk_body(j, _):
            drain_strip(0, j)
            stage_strip(0, j + 1)
            drain_strip(1, j)
            stage_strip(1, j + 1)
            deg_scatter(j)
            return 0
        lax.fori_loop(0, n_chunks - 1, chunk_body, 0)
        for b in range(_SPC):
            drain_strip(b, n_chunks - 1)
        deg_scatter(n_chunks - 1)

        plsc.subcore_barrier()

        pltpu.sync_copy(acc_sh.at[pl.ds(s * frpt, frpt)],
                        agg_out.at[c].at[pl.ds(s * frpt, frpt)])
        pltpu.sync_copy(deg_sh.at[pl.ds(s * rdpt, rdpt)],
                        deg_out.at[pl.ds(c * n_dst + s * rdpt, rdpt)])

    return agg_kernel


def _self_mm(m):
    def body(dst_ref, ws_ref, b_ref, out_ref):
        out_ref[...] = jnp.dot(dst_ref[...], ws_ref[...],
                               preferred_element_type=jnp.float32) + b_ref[...]

    return pl.pallas_call(
        body,
        out_shape=jax.ShapeDtypeStruct((m, D), jnp.float32),
        grid=(1,),
        in_specs=[
            pl.BlockSpec((m, D), lambda i: (0, 0)),
            pl.BlockSpec((D, D), lambda i: (0, 0)),
            pl.BlockSpec((1, D), lambda i: (0, 0)),
        ],
        out_specs=pl.BlockSpec((m, D), lambda i: (0, 0)),
    )


def _neigh_mm(m, relu):
    def body(self_ref, aggl_ref, aggr_ref, deg_ref, wnt_ref, wnb_ref,
             out_ref):
        deg = jnp.maximum(deg_ref[:, 0:1], 1.0)
        acc = self_ref[...]
        acc = acc + jnp.dot(aggl_ref[...] / deg, wnt_ref[...],
                            preferred_element_type=jnp.float32)
        acc = acc + jnp.dot(aggr_ref[...] / deg, wnb_ref[...],
                            preferred_element_type=jnp.float32)
        if relu:
            acc = jnp.maximum(acc, 0.0)
        out_ref[...] = acc

    half = D // _NC
    return pl.pallas_call(
        body,
        out_shape=jax.ShapeDtypeStruct((m, D), jnp.float32),
        grid=(1,),
        in_specs=[
            pl.BlockSpec((m, D), lambda i: (0, 0)),
            pl.BlockSpec((m, half), lambda i: (0, 0)),
            pl.BlockSpec((m, half), lambda i: (0, 0)),
            pl.BlockSpec((m, _DW), lambda i: (0, 0)),
            pl.BlockSpec((half, D), lambda i: (0, 0)),
            pl.BlockSpec((half, D), lambda i: (0, 0)),
        ],
        out_specs=pl.BlockSpec((m, D), lambda i: (0, 0)),
    )


_agg0 = _make_agg(N1, E0)
_agg1 = _make_agg(N2, E1)
_self1 = _self_mm(N1)
_self2 = _self_mm(N2)
_nmm1 = _neigh_mm(N1, relu=True)
_nmm2 = _neigh_mm(N2, relu=False)


def kernel(x, edge_src0, edge_dst0, edge_src1, edge_dst1, num_dst0, num_dst1,
           W_self1, W_neigh1, b1, W_self2, W_neigh2, b2):
    src0 = edge_src0.astype(jnp.int32).reshape(E0 // _CHUNK, _CHUNK)
    dst0 = edge_dst0.astype(jnp.int32).reshape(E0 // _CHUNK, _CHUNK)
    src1 = edge_src1.astype(jnp.int32).reshape(E1 // _CHUNK, _CHUNK)
    dst1 = edge_dst1.astype(jnp.int32).reshape(E1 // _CHUNK, _CHUNK)
    half = D // _NC

    self1 = _self1(x[:N1], W_self1, b1.reshape(1, D))
    agg1, deg1 = _agg0(src0, dst0, x.reshape(N0 * _STRIPS, _SW))
    h = _nmm1(self1, agg1[0].reshape(N1, half), agg1[1].reshape(N1, half),
              deg1[:N1], W_neigh1[:half], W_neigh1[half:])
    self2 = _self2(h[:N2], W_self2, b2.reshape(1, D))
    agg2, deg2 = _agg1(src1, dst1, h.reshape(N1 * _STRIPS, _SW))
    return _nmm2(self2, agg2[0].reshape(N2, half), agg2[1].reshape(N2, half),
                 deg2[:N2], W_neigh2[:half], W_neigh2[half:])


# speedup vs baseline: 5.3093x; 1.1145x over previous
import functools

import jax
import jax.numpy as jnp
from jax import lax
from jax.experimental import pallas as pl
from jax.experimental.pallas import tpu as pltpu
from jax.experimental.pallas import tpu_sc as plsc

N0, N1, N2 = 10000, 2048, 512
E0, E1 = 32768, 8192
D = 512

_NC, _NS = 2, 16
_CHUNK = 128
_SW = 128
_STRIPS = D // _SW
_SPC = _STRIPS // _NC
_ZR = 64
_DW = 128


def _make_agg(n_dst, n_edges, n_src):
    e_per_w = n_edges // _NS
    n_chunks = e_per_w // _CHUNK
    vpc = _CHUNK // 16
    rdpt = n_dst // _NS
    frpt = n_dst * _SPC // _NS
    assert n_chunks * _CHUNK * _NS == n_edges
    assert rdpt * _NS == n_dst

    mesh = plsc.VectorSubcoreMesh(core_axis_name="c", subcore_axis_name="s",
                                  num_cores=_NC, num_subcores=_NS)

    @functools.partial(
        pl.kernel,
        out_type=(
            jax.ShapeDtypeStruct((_NC, n_dst * _SPC, _SW), jnp.float32),
            jax.ShapeDtypeStruct((_NC * n_dst, _DW), jnp.float32),
        ),
        mesh=mesh,
        scratch_types=[
            pltpu.VMEM((n_chunks, _CHUNK), jnp.int32),
            pltpu.VMEM((n_chunks, _CHUNK), jnp.int32),
            pltpu.VMEM((_SPC, _CHUNK), jnp.int32),
            pltpu.VMEM((_CHUNK,), jnp.int32),
            pltpu.VMEM((_SPC, _CHUNK, _SW), jnp.float32),
            pltpu.VMEM((_CHUNK, _DW), jnp.float32),
            pltpu.VMEM((_ZR, _SW), jnp.float32),
            pltpu.VMEM((_ZR, _DW), jnp.float32),
            pltpu.VMEM_SHARED((n_dst * _SPC, _SW), jnp.float32),
            pltpu.VMEM_SHARED((n_dst, _DW), jnp.float32),
            pltpu.SemaphoreType.DMA,
            pltpu.SemaphoreType.DMA,
        ],
    )
    def agg_kernel(src_hbm, dst_hbm, x_hbm, agg_out, deg_out,
                   src_v, dst_v, gidx_v, sidx_v, rows_v, ones_v, zbuf_v,
                   zdeg_v, acc_sh, deg_sh, sem0, sem1):
        c = lax.axis_index("c")
        s = lax.axis_index("s")

        zero = jnp.zeros((16,), jnp.float32)
        one = jnp.ones((16,), jnp.float32)
        vpr = _SW // 16

        def fill_z(i, _):
            zbuf_v[i // vpr, pl.ds((i % vpr) * 16, 16)] = zero
            return 0
        lax.fori_loop(0, _ZR * vpr, fill_z, 0)

        def fill_zd(i, _):
            zdeg_v[i, pl.ds(0, 16)] = zero
            return 0
        lax.fori_loop(0, _ZR, fill_zd, 0)

        pltpu.sync_copy(src_hbm.at[pl.ds(s * n_chunks, n_chunks)], src_v)
        pltpu.sync_copy(dst_hbm.at[pl.ds(s * n_chunks, n_chunks)], dst_v)

        for t in range(0, frpt, _ZR):
            w = min(_ZR, frpt - t)
            pltpu.sync_copy(zbuf_v.at[pl.ds(0, w)],
                            acc_sh.at[pl.ds(s * frpt + t, w)])
        for t in range(0, rdpt, _ZR):
            w = min(_ZR, rdpt - t)
            pltpu.sync_copy(zdeg_v.at[pl.ds(0, w)],
                            deg_sh.at[pl.ds(s * rdpt + t, w)])

        def fill_o(i, _):
            ones_v[i, pl.ds(0, 16)] = one
            return 0
        lax.fori_loop(0, _CHUNK, fill_o, 0)

        plsc.subcore_barrier()

        strip0 = (c * _SPC).astype(jnp.int32)
        sems = (sem0, sem1)

        def stage_strip(b, j):
            for v in range(vpc):
                gidx_v[b, pl.ds(v * 16, 16)] = src_v[j, pl.ds(v * 16, 16)]
            pltpu.async_copy(
                x_hbm.at[gidx_v.at[b], pl.ds((strip0 + b) * _SW, _SW)],
                rows_v.at[b], sems[b])

        def drain_strip(b, j):
            pltpu.make_async_copy(x_hbm.at[pl.ds(0, _CHUNK), pl.ds(0, _SW)],
                                  rows_v.at[b], sems[b]).wait()
            for v in range(vpc):
                sidx_v[pl.ds(v * 16, 16)] = (
                    dst_v[j, pl.ds(v * 16, 16)] * _SPC + b)
            pltpu.sync_copy(rows_v.at[b], acc_sh.at[sidx_v], add=True)

        def deg_scatter(j):
            for v in range(vpc):
                sidx_v[pl.ds(v * 16, 16)] = dst_v[j, pl.ds(v * 16, 16)]
            pltpu.sync_copy(ones_v, deg_sh.at[sidx_v], add=True)

        for b in range(_SPC):
            stage_strip(b, 0)

        def chunk_body(j, _):
            drain_strip(0, j)
            stage_strip(0, j + 1)
            drain_strip(1, j)
            stage_strip(1, j + 1)
            deg_scatter(j)
            return 0
        lax.fori_loop(0, n_chunks - 1, chunk_body, 0)
        for b in range(_SPC):
            drain_strip(b, n_chunks - 1)
        deg_scatter(n_chunks - 1)

        plsc.subcore_barrier()

        pltpu.sync_copy(acc_sh.at[pl.ds(s * frpt, frpt)],
                        agg_out.at[c].at[pl.ds(s * frpt, frpt)])
        pltpu.sync_copy(deg_sh.at[pl.ds(s * rdpt, rdpt)],
                        deg_out.at[pl.ds(c * n_dst + s * rdpt, rdpt)])

    return agg_kernel


def _self_mm(m):
    def body(dst_ref, ws_ref, b_ref, out_ref):
        out_ref[...] = jnp.dot(dst_ref[...], ws_ref[...],
                               preferred_element_type=jnp.float32) + b_ref[...]

    return pl.pallas_call(
        body,
        out_shape=jax.ShapeDtypeStruct((m, D), jnp.float32),
        grid=(1,),
        in_specs=[
            pl.BlockSpec((m, D), lambda i: (0, 0)),
            pl.BlockSpec((D, D), lambda i: (0, 0)),
            pl.BlockSpec((1, D), lambda i: (0, 0)),
        ],
        out_specs=pl.BlockSpec((m, D), lambda i: (0, 0)),
    )


def _neigh_mm(m, relu):
    def body(self_ref, aggl_ref, aggr_ref, deg_ref, wnt_ref, wnb_ref,
             out_ref):
        deg = jnp.maximum(deg_ref[:, 0:1], 1.0)
        acc = self_ref[...]
        acc = acc + jnp.dot(aggl_ref[...] / deg, wnt_ref[...],
                            preferred_element_type=jnp.float32)
        acc = acc + jnp.dot(aggr_ref[...] / deg, wnb_ref[...],
                            preferred_element_type=jnp.float32)
        if relu:
            acc = jnp.maximum(acc, 0.0)
        out_ref[...] = acc

    half = D // _NC
    return pl.pallas_call(
        body,
        out_shape=jax.ShapeDtypeStruct((m, D), jnp.float32),
        grid=(1,),
        in_specs=[
            pl.BlockSpec((m, D), lambda i: (0, 0)),
            pl.BlockSpec((m, half), lambda i: (0, 0)),
            pl.BlockSpec((m, half), lambda i: (0, 0)),
            pl.BlockSpec((m, _DW), lambda i: (0, 0)),
            pl.BlockSpec((half, D), lambda i: (0, 0)),
            pl.BlockSpec((half, D), lambda i: (0, 0)),
        ],
        out_specs=pl.BlockSpec((m, D), lambda i: (0, 0)),
    )


_agg0 = _make_agg(N1, E0, N0)
_agg1 = _make_agg(N2, E1, N1)
_self1 = _self_mm(N1)
_self2 = _self_mm(N2)
_nmm1 = _neigh_mm(N1, relu=True)
_nmm2 = _neigh_mm(N2, relu=False)


def kernel(x, edge_src0, edge_dst0, edge_src1, edge_dst1, num_dst0, num_dst1,
           W_self1, W_neigh1, b1, W_self2, W_neigh2, b2):
    src0 = edge_src0.astype(jnp.int32).reshape(E0 // _CHUNK, _CHUNK)
    dst0 = edge_dst0.astype(jnp.int32).reshape(E0 // _CHUNK, _CHUNK)
    src1 = edge_src1.astype(jnp.int32).reshape(E1 // _CHUNK, _CHUNK)
    dst1 = edge_dst1.astype(jnp.int32).reshape(E1 // _CHUNK, _CHUNK)
    half = D // _NC

    self1 = _self1(x[:N1], W_self1, b1.reshape(1, D))
    agg1, deg1 = _agg0(src0, dst0, x)
    h = _nmm1(self1, agg1[0].reshape(N1, half), agg1[1].reshape(N1, half),
              deg1[:N1], W_neigh1[:half], W_neigh1[half:])
    self2 = _self2(h[:N2], W_self2, b2.reshape(1, D))
    agg2, deg2 = _agg1(src1, dst1, h)
    return _nmm2(self2, agg2[0].reshape(N2, half), agg2[1].reshape(N2, half),
                 deg2[:N2], W_neigh2[:half], W_neigh2[half:])


# speedup vs baseline: 5.9932x; 1.1288x over previous
import functools

import jax
import jax.numpy as jnp
from jax import lax
from jax.experimental import pallas as pl
from jax.experimental.pallas import tpu as pltpu
from jax.experimental.pallas import tpu_sc as plsc

N0, N1, N2 = 10000, 2048, 512
E0, E1 = 32768, 8192
D = 512

_NC, _NS = 2, 16
_CHUNK = 128
_SW = 128
_STRIPS = D // _SW
_SPC = _STRIPS // _NC
_ZR = 64
_DW = 128


def _make_agg(n_dst, n_edges, n_src):
    e_per_w = n_edges // _NS
    n_chunks = e_per_w // _CHUNK
    vpc = _CHUNK // 16
    rdpt = n_dst // _NS
    frpt = n_dst * _SPC // _NS
    assert n_chunks * _CHUNK * _NS == n_edges
    assert rdpt * _NS == n_dst

    mesh = plsc.VectorSubcoreMesh(core_axis_name="c", subcore_axis_name="s",
                                  num_cores=_NC, num_subcores=_NS)

    @functools.partial(
        pl.kernel,
        out_type=(
            jax.ShapeDtypeStruct((_NC, n_dst * _SPC, _SW), jnp.float32),
            jax.ShapeDtypeStruct((_NC * n_dst, _DW), jnp.float32),
        ),
        mesh=mesh,
        scratch_types=[
            pltpu.VMEM((n_chunks, _CHUNK), jnp.int32),
            pltpu.VMEM((n_chunks, _CHUNK), jnp.int32),
            pltpu.VMEM((_SPC, _CHUNK), jnp.int32),
            pltpu.VMEM((_CHUNK,), jnp.int32),
            pltpu.VMEM((_SPC, _CHUNK, _SW), jnp.float32),
            pltpu.VMEM((_CHUNK, _DW), jnp.float32),
            pltpu.VMEM((_ZR, _SW), jnp.float32),
            pltpu.VMEM((_ZR, _DW), jnp.float32),
            pltpu.VMEM_SHARED((n_dst * _SPC, _SW), jnp.float32),
            pltpu.VMEM_SHARED((n_dst, _DW), jnp.float32),
            pltpu.SemaphoreType.DMA,
            pltpu.SemaphoreType.DMA,
        ],
    )
    def agg_kernel(src_hbm, dst_hbm, x_hbm, agg_out, deg_out,
                   src_v, dst_v, gidx_v, sidx_v, rows_v, ones_v, zbuf_v,
                   zdeg_v, acc_sh, deg_sh, sem0, sem1):
        c = lax.axis_index("c")
        s = lax.axis_index("s")

        zero = jnp.zeros((16,), jnp.float32)
        one = jnp.ones((16,), jnp.float32)
        vpr = _SW // 16

        def fill_z(i, _):
            zbuf_v[i // vpr, pl.ds((i % vpr) * 16, 16)] = zero
            return 0
        lax.fori_loop(0, _ZR * vpr, fill_z, 0)

        def fill_zd(i, _):
            zdeg_v[i, pl.ds(0, 16)] = zero
            return 0
        lax.fori_loop(0, _ZR, fill_zd, 0)

        pltpu.sync_copy(src_hbm.at[pl.ds(s * n_chunks, n_chunks)], src_v)
        pltpu.sync_copy(dst_hbm.at[pl.ds(s * n_chunks, n_chunks)], dst_v)

        for t in range(0, frpt, _ZR):
            w = min(_ZR, frpt - t)
            pltpu.sync_copy(zbuf_v.at[pl.ds(0, w)],
                            acc_sh.at[pl.ds(s * frpt + t, w)])
        for t in range(0, rdpt, _ZR):
            w = min(_ZR, rdpt - t)
            pltpu.sync_copy(zdeg_v.at[pl.ds(0, w)],
                            deg_sh.at[pl.ds(s * rdpt + t, w)])

        def fill_o(i, _):
            ones_v[i, pl.ds(0, 16)] = one
            return 0
        lax.fori_loop(0, _CHUNK, fill_o, 0)

        plsc.subcore_barrier()

        strip0 = (c * _SPC).astype(jnp.int32)
        sems = (sem0, sem1)

        def stage_strip(b, j):
            for v in range(vpc):
                gidx_v[b, pl.ds(v * 16, 16)] = src_v[j, pl.ds(v * 16, 16)]
            pltpu.async_copy(
                x_hbm.at[gidx_v.at[b], pl.ds((strip0 + b) * _SW, _SW)],
                rows_v.at[b], sems[b])

        def drain_strip(b, j):
            pltpu.make_async_copy(x_hbm.at[pl.ds(0, _CHUNK), pl.ds(0, _SW)],
                                  rows_v.at[b], sems[b]).wait()
            for v in range(vpc):
                sidx_v[pl.ds(v * 16, 16)] = (
                    dst_v[j, pl.ds(v * 16, 16)] + b * n_dst)
            pltpu.sync_copy(rows_v.at[b], acc_sh.at[sidx_v], add=True)

        def deg_scatter(j):
            for v in range(vpc):
                sidx_v[pl.ds(v * 16, 16)] = dst_v[j, pl.ds(v * 16, 16)]
            pltpu.sync_copy(ones_v, deg_sh.at[sidx_v], add=True)

        for b in range(_SPC):
            stage_strip(b, 0)

        def chunk_body(j, _):
            drain_strip(0, j)
            stage_strip(0, j + 1)
            drain_strip(1, j)
            stage_strip(1, j + 1)
            deg_scatter(j)
            return 0
        lax.fori_loop(0, n_chunks - 1, chunk_body, 0)
        for b in range(_SPC):
            drain_strip(b, n_chunks - 1)
        deg_scatter(n_chunks - 1)

        plsc.subcore_barrier()

        pltpu.sync_copy(acc_sh.at[pl.ds(s * frpt, frpt)],
                        agg_out.at[c].at[pl.ds(s * frpt, frpt)])
        pltpu.sync_copy(deg_sh.at[pl.ds(s * rdpt, rdpt)],
                        deg_out.at[pl.ds(c * n_dst + s * rdpt, rdpt)])

    return agg_kernel


def _self_mm(m):
    def body(dst_ref, ws_ref, b_ref, out_ref):
        out_ref[...] = jnp.dot(dst_ref[...], ws_ref[...],
                               preferred_element_type=jnp.float32) + b_ref[...]

    return pl.pallas_call(
        body,
        out_shape=jax.ShapeDtypeStruct((m, D), jnp.float32),
        grid=(1,),
        in_specs=[
            pl.BlockSpec((m, D), lambda i: (0, 0)),
            pl.BlockSpec((D, D), lambda i: (0, 0)),
            pl.BlockSpec((1, D), lambda i: (0, 0)),
        ],
        out_specs=pl.BlockSpec((m, D), lambda i: (0, 0)),
    )


def _neigh_mm(m, relu):
    def body(self_ref, agg_ref, deg_ref, wn_ref, out_ref):
        acc = jnp.zeros((m, D), jnp.float32)
        for c in range(_NC):
            for b in range(_SPC):
                k = c * _SPC + b
                acc = acc + jnp.dot(
                    agg_ref[c, b * m:(b + 1) * m, :],
                    wn_ref[k * _SW:(k + 1) * _SW, :],
                    preferred_element_type=jnp.float32)
        deg = jnp.maximum(deg_ref[:, 0:1], 1.0)
        acc = self_ref[...] + acc / deg
        if relu:
            acc = jnp.maximum(acc, 0.0)
        out_ref[...] = acc

    return pl.pallas_call(
        body,
        out_shape=jax.ShapeDtypeStruct((m, D), jnp.float32),
        grid=(1,),
        in_specs=[
            pl.BlockSpec((m, D), lambda i: (0, 0)),
            pl.BlockSpec((_NC, _SPC * m, _SW), lambda i: (0, 0, 0)),
            pl.BlockSpec((m, _DW), lambda i: (0, 0)),
            pl.BlockSpec((D, D), lambda i: (0, 0)),
        ],
        out_specs=pl.BlockSpec((m, D), lambda i: (0, 0)),
    )


_agg0 = _make_agg(N1, E0, N0)
_agg1 = _make_agg(N2, E1, N1)
_self1 = _self_mm(N1)
_self2 = _self_mm(N2)
_nmm1 = _neigh_mm(N1, relu=True)
_nmm2 = _neigh_mm(N2, relu=False)


def kernel(x, edge_src0, edge_dst0, edge_src1, edge_dst1, num_dst0, num_dst1,
           W_self1, W_neigh1, b1, W_self2, W_neigh2, b2):
    src0 = edge_src0.astype(jnp.int32).reshape(E0 // _CHUNK, _CHUNK)
    dst0 = edge_dst0.astype(jnp.int32).reshape(E0 // _CHUNK, _CHUNK)
    src1 = edge_src1.astype(jnp.int32).reshape(E1 // _CHUNK, _CHUNK)
    dst1 = edge_dst1.astype(jnp.int32).reshape(E1 // _CHUNK, _CHUNK)
    self1 = _self1(x, W_self1, b1.reshape(1, D))
    agg1, deg1 = _agg0(src0, dst0, x)
    h = _nmm1(self1, agg1, deg1, W_neigh1)
    self2 = _self2(h, W_self2, b2.reshape(1, D))
    agg2, deg2 = _agg1(src1, dst1, h)
    return _nmm2(self2, agg2, deg2, W_neigh2)
